# Initial kernel scaffold; baseline (speedup 1.0000x reference)
import functools

import jax
import jax.numpy as jnp
from jax import lax
from jax.experimental import pallas as pl
from jax.experimental.pallas import tpu as pltpu
from jax.experimental.pallas import tpu_sc as plsc

N = 10000
E = 320000
IN_DIM = 5
H = 128
B = 16
NCLS = 5

NC = 2
NS = 16
NW = NC * NS
EPT = E // NW
K = 80
NCHUNK = EPT // K
NPT = N // NS
ZR = 125
RB = 1000
GRID = N // RB
RSQRT_1PEPS = float(1.0 / (1.0 + 1e-5) ** 0.5)

_mesh = plsc.VectorSubcoreMesh(core_axis_name="c", subcore_axis_name="s")


@functools.partial(
    pl.kernel,
    out_type=jax.ShapeDtypeStruct((NC, N, 16), jnp.float32),
    mesh=_mesh,
    scratch_types=[
        pltpu.VMEM((K,), jnp.int32),
        pltpu.VMEM((K, 16), jnp.float32),
        pltpu.VMEM((ZR, 16), jnp.float32),
        pltpu.VMEM_SHARED((N, 16), jnp.float32),
    ],
)
def _sc_degree(col_hbm, ones_hbm, zeros_hbm, out_hbm, col_v, ones_v, zeros_v, acc):
    c = lax.axis_index("c")
    s = lax.axis_index("s")
    wid = s * NC + c
    pltpu.sync_copy(zeros_hbm, zeros_v)
    pltpu.sync_copy(ones_hbm, ones_v)
    for j in range(NPT // ZR):
        pltpu.sync_copy(zeros_v, acc.at[pl.ds(s * NPT + j * ZR, ZR)])
    plsc.subcore_barrier()

    def body(i, carry):
        off = wid * EPT + i * K
        pltpu.sync_copy(col_hbm.at[pl.ds(off, K)], col_v)
        pltpu.sync_copy(ones_v, acc.at[col_v], add=True)
        return carry

    lax.fori_loop(0, NCHUNK, body, 0)
    plsc.subcore_barrier()
    for j in range(NPT // ZR):
        sl = pl.ds(s * NPT + j * ZR, ZR)
        pltpu.sync_copy(acc.at[sl], out_hbm.at[c, sl])


@functools.partial(
    pl.kernel,
    out_type=jax.ShapeDtypeStruct((NC, N, H), jnp.float32),
    mesh=_mesh,
    scratch_types=[
        pltpu.VMEM((K,), jnp.int32),
        pltpu.VMEM((K,), jnp.int32),
        pltpu.VMEM((K, H), jnp.float32),
        pltpu.VMEM((ZR, H), jnp.float32),
        pltpu.VMEM_SHARED((N, H), jnp.float32),
        pltpu.SemaphoreType.DMA,
    ],
)
def _sc_edge_agg(mp_hbm, row_hbm, col_hbm, zeros_hbm, out_hbm,
                 row_v, col_v, rows_v, zeros_v, acc, sem):
    c = lax.axis_index("c")
    s = lax.axis_index("s")
    wid = s * NC + c
    pltpu.sync_copy(zeros_hbm, zeros_v)
    for j in range(NPT // ZR):
        pltpu.sync_copy(zeros_v, acc.at[pl.ds(s * NPT + j * ZR, ZR)])
    plsc.subcore_barrier()

    def body(i, carry):
        off = wid * EPT + i * K
        pltpu.sync_copy(row_hbm.at[pl.ds(off, K)], row_v)
        pltpu.sync_copy(col_hbm.at[pl.ds(off, K)], col_v)
        pltpu.async_copy(mp_hbm.at[row_v], rows_v, sem).wait()
        pltpu.sync_copy(rows_v, acc.at[col_v], add=True)
        return carry

    lax.fori_loop(0, NCHUNK, body, 0)
    plsc.subcore_barrier()
    for j in range(NPT // ZR):
        sl = pl.ds(s * NPT + j * ZR, ZR)
        pltpu.sync_copy(acc.at[sl], out_hbm.at[c, sl])


def _tc_input_body(x_ref, d0_ref, d1_ref, w_in_ref, b_in_ref, ws0_ref,
                   m0_ref, dinv_ref):
    deg = 1.0 + d0_ref[:, 0:1] + d1_ref[:, 0:1]
    dinv = lax.rsqrt(deg)
    h0 = jnp.dot(x_ref[...], w_in_ref[...],
                 preferred_element_type=jnp.float32) + b_in_ref[...]
    m0_ref[...] = jnp.dot(dinv * h0, ws0_ref[...],
                          preferred_element_type=jnp.float32)
    dinv_ref[...] = dinv


def _make_input_call(interpret=False):
    return pl.pallas_call(
        _tc_input_body,
        grid=(GRID,),
        in_specs=[
            pl.BlockSpec((RB, IN_DIM), lambda i: (i, 0)),
            pl.BlockSpec((RB, 16), lambda i: (i, 0)),
            pl.BlockSpec((RB, 16), lambda i: (i, 0)),
            pl.BlockSpec((IN_DIM, H), lambda i: (0, 0)),
            pl.BlockSpec((1, H), lambda i: (0, 0)),
            pl.BlockSpec((H, H), lambda i: (0, 0)),
        ],
        out_specs=[pl.BlockSpec((RB, H), lambda i: (i, 0)),
                   pl.BlockSpec((RB, 1), lambda i: (i, 0))],
        out_shape=[jax.ShapeDtypeStruct((N, H), jnp.float32),
                   jax.ShapeDtypeStruct((N, 1), jnp.float32)],
        interpret=interpret,
    )


def _tc_layer_body(p0_ref, p1_ref, m_ref, dinv_ref, g_ref, be_ref, bs_ref,
                   w_ref, out_ref):
    dinv = dinv_ref[...]
    seg = dinv * (p0_ref[...] + p1_ref[...] + m_ref[...])
    c1 = g_ref[...] * RSQRT_1PEPS
    h = jnp.maximum(seg * c1 + (bs_ref[...] * c1 + be_ref[...]), 0.0)
    out_ref[...] = jnp.dot(dinv * h, w_ref[...],
                           preferred_element_type=jnp.float32)


def _make_layer_call(interpret=False):
    return pl.pallas_call(
        _tc_layer_body,
        grid=(GRID,),
        in_specs=[
            pl.BlockSpec((RB, H), lambda i: (i, 0)),
            pl.BlockSpec((RB, H), lambda i: (i, 0)),
            pl.BlockSpec((RB, H), lambda i: (i, 0)),
            pl.BlockSpec((RB, 1), lambda i: (i, 0)),
            pl.BlockSpec((1, H), lambda i: (0, 0)),
            pl.BlockSpec((1, H), lambda i: (0, 0)),
            pl.BlockSpec((1, H), lambda i: (0, 0)),
            pl.BlockSpec((H, H), lambda i: (0, 0)),
        ],
        out_specs=pl.BlockSpec((RB, H), lambda i: (i, 0)),
        out_shape=jax.ShapeDtypeStruct((N, H), jnp.float32),
        interpret=interpret,
    )


def _tc_pool_body(p0_ref, p1_ref, m_ref, dinv_ref, g_ref, be_ref, bs_ref,
                  batch_ref, wc1_ref, bc1_ref, wc2_ref, bc2_ref, wc3_ref,
                  bc3_ref, out_ref, sum_acc, max_acc, cnt_acc):
    pid = pl.program_id(0)

    @pl.when(pid == 0)
    def _():
        sum_acc[...] = jnp.zeros_like(sum_acc)
        max_acc[...] = jnp.full_like(max_acc, -1.0)
        cnt_acc[...] = jnp.zeros_like(cnt_acc)

    dinv = dinv_ref[...]
    seg = dinv * (p0_ref[...] + p1_ref[...] + m_ref[...])
    c1 = g_ref[...] * RSQRT_1PEPS
    h = jnp.maximum(seg * c1 + (bs_ref[...] * c1 + be_ref[...]), 0.0)

    b_ids = batch_ref[...]
    onehot = (b_ids == lax.broadcasted_iota(jnp.int32, (1, B), 1)
              ).astype(jnp.float32)
    sum_acc[...] += lax.dot_general(onehot, h, (((0,), (0,)), ((), ())),
                                    preferred_element_type=jnp.float32)
    cnt_acc[...] += jnp.broadcast_to(jnp.sum(onehot, axis=0)[:, None], (B, H))
    for b in range(B):
        mb = jnp.where(b_ids == b, h, -1.0)
        max_acc[b:b + 1, :] = jnp.maximum(max_acc[b:b + 1, :],
                                          jnp.max(mb, axis=0, keepdims=True))

    @pl.when(pid == GRID - 1)
    def _():
        cnt = cnt_acc[...]
        mean_p = sum_acc[...] / jnp.maximum(cnt, 1.0)
        max_p = jnp.where(cnt > 0, max_acc[...], 0.0)
        pooled = jnp.concatenate([mean_p, max_p], axis=1)
        z = jnp.maximum(jnp.dot(pooled, wc1_ref[...],
                                preferred_element_type=jnp.float32)
                        + bc1_ref[...], 0.0)
        z = jnp.maximum(jnp.dot(z, wc2_ref[...],
                                preferred_element_type=jnp.float32)
                        + bc2_ref[...], 0.0)
        out_ref[...] = jnp.dot(z, wc3_ref[...],
                               preferred_element_type=jnp.float32) + bc3_ref[...]


def _make_pool_call(interpret=False):
    return pl.pallas_call(
        _tc_pool_body,
        grid=(GRID,),
        in_specs=[
            pl.BlockSpec((RB, H), lambda i: (i, 0)),
            pl.BlockSpec((RB, H), lambda i: (i, 0)),
            pl.BlockSpec((RB, H), lambda i: (i, 0)),
            pl.BlockSpec((RB, 1), lambda i: (i, 0)),
            pl.BlockSpec((1, H), lambda i: (0, 0)),
            pl.BlockSpec((1, H), lambda i: (0, 0)),
            pl.BlockSpec((1, H), lambda i: (0, 0)),
            pl.BlockSpec((RB, 1), lambda i: (i, 0)),
            pl.BlockSpec((2 * H, H), lambda i: (0, 0)),
            pl.BlockSpec((1, H), lambda i: (0, 0)),
            pl.BlockSpec((H, H // 2), lambda i: (0, 0)),
            pl.BlockSpec((1, H // 2), lambda i: (0, 0)),
            pl.BlockSpec((H // 2, NCLS), lambda i: (0, 0)),
            pl.BlockSpec((1, NCLS), lambda i: (0, 0)),
        ],
        out_specs=pl.BlockSpec((B, NCLS), lambda i: (0, 0)),
        out_shape=jax.ShapeDtypeStruct((B, NCLS), jnp.float32),
        scratch_shapes=[
            pltpu.VMEM((B, H), jnp.float32),
            pltpu.VMEM((B, H), jnp.float32),
            pltpu.VMEM((B, H), jnp.float32),
        ],
        interpret=interpret,
    )


_input_call = _make_input_call()
_layer_call = _make_layer_call()
_pool_call = _make_pool_call()


def kernel(x, edge_index, batch, W_in, b_in, Ws, bs, gammas, betas,
           Wc1, bc1, Wc2, bc2, Wc3, bc3):
    row = edge_index[0]
    col = edge_index[1]
    ones16 = jnp.ones((K, 16), jnp.float32)
    zeros16 = jnp.zeros((ZR, 16), jnp.float32)
    zerosH = jnp.zeros((ZR, H), jnp.float32)

    deg_parts = _sc_degree(col, ones16, zeros16)
    m, dinv = _input_call(x, deg_parts[0], deg_parts[1],
                          W_in, b_in.reshape(1, H), Ws[0])
    for i in range(3):
        parts = _sc_edge_agg(m, row, col, zerosH)
        m = _layer_call(parts[0], parts[1], m, dinv,
                        gammas[i].reshape(1, H), betas[i].reshape(1, H),
                        bs[i].reshape(1, H), Ws[i + 1])
    parts = _sc_edge_agg(m, row, col, zerosH)
    out = _pool_call(parts[0], parts[1], m, dinv,
                     gammas[3].reshape(1, H), betas[3].reshape(1, H),
                     bs[3].reshape(1, H), batch.reshape(N, 1),
                     Wc1, bc1.reshape(1, H), Wc2, bc2.reshape(1, H // 2),
                     Wc3, bc3.reshape(1, NCLS))
    return out


# speedup vs baseline: 22.0660x; 22.0660x over previous
import functools

import jax
import jax.numpy as jnp
from jax import lax
from jax.experimental import pallas as pl
from jax.experimental.pallas import tpu as pltpu
from jax.experimental.pallas import tpu_sc as plsc

N = 10000
E = 320000
IN_DIM = 5
H = 128
B = 16
NCLS = 5

NC = 2
NS = 16
NW = NC * NS
EPT = E // NW
K = 80
NCHUNK = EPT // K
N_PAD = 10240
NPT = N_PAD // NS
ZR = 16
NBUF = 3
PI = 16
KD = 80
NCHUNKD = EPT // KD
RB = 1000
GRID = N // RB
RSQRT_1PEPS = float(1.0 / (1.0 + 1e-5) ** 0.5)


@functools.lru_cache(maxsize=None)
def _sc_kernels():
    mesh = plsc.VectorSubcoreMesh(core_axis_name="c", subcore_axis_name="s")
    deg = functools.partial(
        pl.kernel,
        out_type=jax.ShapeDtypeStruct((NC, N_PAD, H), jnp.float32),
        mesh=mesh,
        scratch_types=[
            pltpu.VMEM((PI, KD), jnp.int32),
            pltpu.VMEM((KD, H), jnp.float32),
            pltpu.VMEM((ZR, H), jnp.float32),
            pltpu.VMEM_SHARED((N_PAD, H), jnp.float32),
            pltpu.SemaphoreType.DMA((PI,)),
            pltpu.SemaphoreType.DMA((NBUF,)),
        ],
    )(_sc_degree_body)
    agg = functools.partial(
        pl.kernel,
        out_type=jax.ShapeDtypeStruct((NC, N_PAD, H), jnp.float32),
        mesh=mesh,
        scratch_types=[
            pltpu.VMEM((PI, K), jnp.int32),
            pltpu.VMEM((PI, K), jnp.int32),
            pltpu.VMEM((NBUF, K, H), jnp.float32),
            pltpu.VMEM((ZR, H), jnp.float32),
            pltpu.VMEM_SHARED((N_PAD, H), jnp.float32),
            pltpu.SemaphoreType.DMA((PI,)),
            pltpu.SemaphoreType.DMA((NBUF,)),
            pltpu.SemaphoreType.DMA((NBUF,)),
        ],
    )(_sc_edge_agg_body)
    return deg, agg


def _sc_degree(col, ones, zeros):
    return _sc_kernels()[0](col, ones, zeros)


def _sc_edge_agg(mp, row, col, zerosH):
    return _sc_kernels()[1](mp, row, col, zerosH)


def _sc_degree_body(col_hbm, ones_hbm, zeros_hbm, out_hbm,
                    colb, ones_v, zeros_v, acc, isem, ssem):
    c = lax.axis_index("c")
    s = lax.axis_index("s")
    wid = s * NC + c
    base = wid * EPT
    pltpu.sync_copy(ones_hbm, ones_v)
    pltpu.sync_copy(zeros_hbm, zeros_v)
    for j in range(NPT // ZR):
        sl = pl.ds(s * NPT + j * ZR, ZR)
        pltpu.async_copy(zeros_v, acc.at[sl], ssem.at[0])
    for j in range(NPT // ZR):
        sl = pl.ds(s * NPT + j * ZR, ZR)
        pltpu.make_async_copy(zeros_v, acc.at[sl], ssem.at[0]).wait()
    plsc.subcore_barrier()

    def fire_idx(ch):
        pltpu.async_copy(col_hbm.at[pl.ds(base + ch * KD, KD)],
                         colb.at[lax.rem(ch, PI)], isem.at[lax.rem(ch, PI)])

    def wait_idx(ch):
        pltpu.make_async_copy(col_hbm.at[pl.ds(base + ch * KD, KD)],
                              colb.at[lax.rem(ch, PI)],
                              isem.at[lax.rem(ch, PI)]).wait()

    def fire_scat(ch):
        pltpu.async_copy(ones_v, acc.at[colb.at[lax.rem(ch, PI)]],
                         ssem.at[lax.rem(ch, NBUF)], add=True)

    def wait_scat(ch):
        pltpu.make_async_copy(ones_v, acc.at[colb.at[lax.rem(ch, PI)]],
                              ssem.at[lax.rem(ch, NBUF)]).wait()

    for ch in range(8):
        fire_idx(ch)

    def body(ch, carry):
        @pl.when(ch + 8 < NCHUNKD)
        def _():
            fire_idx(ch + 8)
        wait_idx(ch)
        fire_scat(ch)
        @pl.when(ch >= NBUF)
        def _():
            wait_scat(ch - NBUF)
        return carry

    lax.fori_loop(0, NCHUNKD, body, 0)
    for ch in range(NCHUNKD - NBUF, NCHUNKD):
        wait_scat(ch)
    plsc.subcore_barrier()
    tile_rows = pl.ds(s * NPT, NPT)
    pltpu.sync_copy(acc.at[tile_rows], out_hbm.at[c, tile_rows])


def _sc_edge_agg_body(mp_hbm, row_hbm, col_hbm, zeros_hbm, out_hbm,
                      rowb, colb, bufs, zeros_v, acc, isem, gsem, ssem):
    c = lax.axis_index("c")
    s = lax.axis_index("s")
    wid = s * NC + c
    base = wid * EPT
    tile_rows = pl.ds(s * NPT, NPT)

    @pl.when(c == 0)
    def _():
        pltpu.sync_copy(mp_hbm.at[tile_rows], acc.at[tile_rows])

    @pl.when(c == 1)
    def _():
        pltpu.sync_copy(zeros_hbm, zeros_v)
        for j in range(NPT // ZR):
            sl = pl.ds(s * NPT + j * ZR, ZR)
            pltpu.async_copy(zeros_v, acc.at[sl], gsem.at[0])
        for j in range(NPT // ZR):
            sl = pl.ds(s * NPT + j * ZR, ZR)
            pltpu.make_async_copy(zeros_v, acc.at[sl], gsem.at[0]).wait()

    plsc.subcore_barrier()

    def fire_idx(ch):
        sl = lax.rem(ch, PI)
        pltpu.async_copy(row_hbm.at[pl.ds(base + ch * K, K)],
                         rowb.at[sl], isem.at[sl])
        pltpu.async_copy(col_hbm.at[pl.ds(base + ch * K, K)],
                         colb.at[sl], isem.at[sl])

    def wait_idx(ch):
        sl = lax.rem(ch, PI)
        pltpu.make_async_copy(row_hbm.at[pl.ds(base + ch * K, K)],
                              rowb.at[sl], isem.at[sl]).wait()
        pltpu.make_async_copy(col_hbm.at[pl.ds(base + ch * K, K)],
                              colb.at[sl], isem.at[sl]).wait()

    def fire_gather(ch):
        pltpu.async_copy(mp_hbm.at[rowb.at[lax.rem(ch, PI)]],
                         bufs.at[lax.rem(ch, NBUF)],
                         gsem.at[lax.rem(ch, NBUF)])

    def wait_gather(ch):
        pltpu.make_async_copy(mp_hbm.at[rowb.at[lax.rem(ch, PI)]],
                              bufs.at[lax.rem(ch, NBUF)],
                              gsem.at[lax.rem(ch, NBUF)]).wait()

    def fire_scat(ch):
        pltpu.async_copy(bufs.at[lax.rem(ch, NBUF)],
                         acc.at[colb.at[lax.rem(ch, PI)]],
                         ssem.at[lax.rem(ch, NBUF)], add=True)

    def wait_scat(ch):
        pltpu.make_async_copy(bufs.at[lax.rem(ch, NBUF)],
                              acc.at[colb.at[lax.rem(ch, PI)]],
                              ssem.at[lax.rem(ch, NBUF)]).wait()

    for ch in range(8):
        fire_idx(ch)
    wait_idx(0)
    fire_gather(0)

    def body(ch, carry):
        @pl.when(ch + 8 < NCHUNK)
        def _():
            fire_idx(ch + 8)

        @pl.when(ch + 1 < NCHUNK)
        def _():
            wait_idx(ch + 1)
            @pl.when(ch >= 2)
            def _():
                wait_scat(ch - 2)
            fire_gather(ch + 1)

        wait_gather(ch)
        fire_scat(ch)
        return carry

    lax.fori_loop(0, NCHUNK, body, 0)
    for ch in range(NCHUNK - NBUF + 1, NCHUNK):
        wait_scat(ch)
    plsc.subcore_barrier()
    pltpu.sync_copy(acc.at[tile_rows], out_hbm.at[c, tile_rows])


def _tc_t0_body(x_ref, w_in_ref, b_in_ref, ws0_ref, t0_ref):
    h0 = jnp.dot(x_ref[...], w_in_ref[...],
                 preferred_element_type=jnp.float32) + b_in_ref[...]
    t0_ref[...] = jnp.dot(h0, ws0_ref[...],
                          preferred_element_type=jnp.float32)


def _make_t0_call(interpret=False):
    return pl.pallas_call(
        _tc_t0_body,
        grid=(GRID,),
        in_specs=[
            pl.BlockSpec((RB, IN_DIM), lambda i: (i, 0)),
            pl.BlockSpec((IN_DIM, H), lambda i: (0, 0)),
            pl.BlockSpec((1, H), lambda i: (0, 0)),
            pl.BlockSpec((H, H), lambda i: (0, 0)),
        ],
        out_specs=pl.BlockSpec((RB, H), lambda i: (i, 0)),
        out_shape=jax.ShapeDtypeStruct((N_PAD, H), jnp.float32),
        interpret=interpret,
    )


def _tc_input_body(d0_ref, d1_ref, t0_ref, m0_ref, dinv_ref):
    deg = 1.0 + d0_ref[:, 0:1] + d1_ref[:, 0:1]
    dinv = lax.rsqrt(deg)
    m0_ref[...] = dinv * t0_ref[...]
    dinv_ref[...] = dinv


def _make_input_call(interpret=False):
    return pl.pallas_call(
        _tc_input_body,
        grid=(GRID,),
        in_specs=[
            pl.BlockSpec((RB, H), lambda i: (i, 0)),
            pl.BlockSpec((RB, H), lambda i: (i, 0)),
            pl.BlockSpec((RB, H), lambda i: (i, 0)),
        ],
        out_specs=[pl.BlockSpec((RB, H), lambda i: (i, 0)),
                   pl.BlockSpec((RB, 1), lambda i: (i, 0))],
        out_shape=[jax.ShapeDtypeStruct((N_PAD, H), jnp.float32),
                   jax.ShapeDtypeStruct((N, 1), jnp.float32)],
        interpret=interpret,
    )


def _tc_layer_body(p0_ref, p1_ref, dinv_ref, g_ref, be_ref, bs_ref,
                   w_ref, out_ref):
    dinv = dinv_ref[...]
    seg = dinv * (p0_ref[...] + p1_ref[...])
    c1 = g_ref[...] * RSQRT_1PEPS
    h = jnp.maximum(seg * c1 + (bs_ref[...] * c1 + be_ref[...]), 0.0)
    out_ref[...] = jnp.dot(dinv * h, w_ref[...],
                           preferred_element_type=jnp.float32)


def _make_layer_call(interpret=False):
    return pl.pallas_call(
        _tc_layer_body,
        grid=(GRID,),
        in_specs=[
            pl.BlockSpec((RB, H), lambda i: (i, 0)),
            pl.BlockSpec((RB, H), lambda i: (i, 0)),
            pl.BlockSpec((RB, 1), lambda i: (i, 0)),
            pl.BlockSpec((1, H), lambda i: (0, 0)),
            pl.BlockSpec((1, H), lambda i: (0, 0)),
            pl.BlockSpec((1, H), lambda i: (0, 0)),
            pl.BlockSpec((H, H), lambda i: (0, 0)),
        ],
        out_specs=pl.BlockSpec((RB, H), lambda i: (i, 0)),
        out_shape=jax.ShapeDtypeStruct((N_PAD, H), jnp.float32),
        interpret=interpret,
    )


def _tc_pool_body(p0_ref, p1_ref, dinv_ref, g_ref, be_ref, bs_ref,
                  batch_ref, wc1_ref, bc1_ref, wc2_ref, bc2_ref, wc3_ref,
                  bc3_ref, out_ref, sum_acc, max_acc, cnt_acc):
    pid = pl.program_id(0)

    @pl.when(pid == 0)
    def _():
        sum_acc[...] = jnp.zeros_like(sum_acc)
        max_acc[...] = jnp.full_like(max_acc, -1.0)
        cnt_acc[...] = jnp.zeros_like(cnt_acc)

    dinv = dinv_ref[...]
    seg = dinv * (p0_ref[...] + p1_ref[...])
    c1 = g_ref[...] * RSQRT_1PEPS
    h = jnp.maximum(seg * c1 + (bs_ref[...] * c1 + be_ref[...]), 0.0)

    b_ids = batch_ref[...]
    onehot = (b_ids == lax.broadcasted_iota(jnp.int32, (1, B), 1)
              ).astype(jnp.float32)
    sum_acc[...] += lax.dot_general(onehot, h, (((0,), (0,)), ((), ())),
                                    preferred_element_type=jnp.float32)
    cnt_acc[...] += jnp.broadcast_to(jnp.sum(onehot, axis=0)[:, None], (B, H))
    for b in range(B):
        mb = jnp.where(b_ids == b, h, -1.0)
        max_acc[b:b + 1, :] = jnp.maximum(max_acc[b:b + 1, :],
                                          jnp.max(mb, axis=0, keepdims=True))

    @pl.when(pid == GRID - 1)
    def _():
        cnt = cnt_acc[...]
        mean_p = sum_acc[...] / jnp.maximum(cnt, 1.0)
        max_p = jnp.where(cnt > 0, max_acc[...], 0.0)
        pooled = jnp.concatenate([mean_p, max_p], axis=1)
        z = jnp.maximum(jnp.dot(pooled, wc1_ref[...],
                                preferred_element_type=jnp.float32)
                        + bc1_ref[...], 0.0)
        z = jnp.maximum(jnp.dot(z, wc2_ref[...],
                                preferred_element_type=jnp.float32)
                        + bc2_ref[...], 0.0)
        out_ref[...] = jnp.dot(z, wc3_ref[...],
                               preferred_element_type=jnp.float32) + bc3_ref[...]


def _make_pool_call(interpret=False):
    return pl.pallas_call(
        _tc_pool_body,
        grid=(GRID,),
        in_specs=[
            pl.BlockSpec((RB, H), lambda i: (i, 0)),
            pl.BlockSpec((RB, H), lambda i: (i, 0)),
            pl.BlockSpec((RB, 1), lambda i: (i, 0)),
            pl.BlockSpec((1, H), lambda i: (0, 0)),
            pl.BlockSpec((1, H), lambda i: (0, 0)),
            pl.BlockSpec((1, H), lambda i: (0, 0)),
            pl.BlockSpec((RB, 1), lambda i: (i, 0)),
            pl.BlockSpec((2 * H, H), lambda i: (0, 0)),
            pl.BlockSpec((1, H), lambda i: (0, 0)),
            pl.BlockSpec((H, H // 2), lambda i: (0, 0)),
            pl.BlockSpec((1, H // 2), lambda i: (0, 0)),
            pl.BlockSpec((H // 2, NCLS), lambda i: (0, 0)),
            pl.BlockSpec((1, NCLS), lambda i: (0, 0)),
        ],
        out_specs=pl.BlockSpec((B, NCLS), lambda i: (0, 0)),
        out_shape=jax.ShapeDtypeStruct((B, NCLS), jnp.float32),
        scratch_shapes=[
            pltpu.VMEM((B, H), jnp.float32),
            pltpu.VMEM((B, H), jnp.float32),
            pltpu.VMEM((B, H), jnp.float32),
        ],
        interpret=interpret,
    )


_t0_call = _make_t0_call()
_input_call = _make_input_call()
_layer_call = _make_layer_call()
_pool_call = _make_pool_call()


def kernel(x, edge_index, batch, W_in, b_in, Ws, bs, gammas, betas,
           Wc1, bc1, Wc2, bc2, Wc3, bc3):
    row = edge_index[0]
    col = edge_index[1]

    zerosH = jnp.zeros((ZR, H), jnp.float32)
    onesKH = jnp.ones((KD, H), jnp.float32)

    t0 = _t0_call(x, W_in, b_in.reshape(1, H), Ws[0])
    deg_parts = _sc_degree(col, onesKH, zerosH)
    m, dinv = _input_call(deg_parts[0], deg_parts[1], t0)
    for i in range(3):
        parts = _sc_edge_agg(m, row, col, zerosH)
        m = _layer_call(parts[0], parts[1], dinv,
                        gammas[i].reshape(1, H), betas[i].reshape(1, H),
                        bs[i].reshape(1, H), Ws[i + 1])
    parts = _sc_edge_agg(m, row, col, zerosH)
    out = _pool_call(parts[0], parts[1], dinv,
                     gammas[3].reshape(1, H), betas[3].reshape(1, H),
                     bs[3].reshape(1, H), batch.reshape(N, 1),
                     Wc1, bc1.reshape(1, H), Wc2, bc2.reshape(1, H // 2),
                     Wc3, bc3.reshape(1, NCLS))
    return out
